# baseline (speedup 1.0000x reference)
import functools

import jax
import jax.numpy as jnp
from jax import lax
from jax.experimental import pallas as pl
from jax.experimental.pallas import tpu as pltpu
from jax.experimental.pallas import tpu_sc as plsc

N = 10000
E = 320000
D_IN = 128
EMB = 64
HALF = EMB // 2

NC, NS = 2, 16
NW = NC * NS
NP = 10240
EP = 327680
CH = 128
RPT = NP // NS
EPT = EP // NS

_sc_mesh = plsc.VectorSubcoreMesh(
    core_axis_name="c", subcore_axis_name="s", num_cores=NC, num_subcores=NS)
_sc_params = pltpu.CompilerParams(use_tc_tiling_on_sc=False)


def _fill(ref, rows, cols, value):
  @pl.loop(0, rows)
  def _(r):
    for k in range(cols // 16):
      ref[r, pl.ds(k * 16, 16)] = jnp.full((16,), value, jnp.float32)


_DCH = EP // NW // CH
_K = 8


def _deg_body(dst2d_hbm, out_hbm, ones_v, stage_v, idx_v, sem, deg_sh):
  c = lax.axis_index("c")
  s = lax.axis_index("s")
  wid = c * NS + s
  _fill(ones_v, CH, 16, 1.0)
  _fill(stage_v, RPT, 16, 0.0)
  zb = s * RPT
  pltpu.sync_copy(stage_v, deg_sh.at[pl.ds(zb, RPT)])
  pltpu.sync_copy(dst2d_hbm.at[pl.ds(wid * _DCH, _DCH)], idx_v)
  plsc.subcore_barrier()
  @pl.loop(0, _DCH)
  def _(j):
    @pl.when(j >= _K)
    def _():
      pltpu.make_async_copy(ones_v, deg_sh.at[idx_v.at[j]], sem).wait()
    pltpu.async_copy(ones_v, deg_sh.at[idx_v.at[j]], sem, add=True)
  for _ in range(_K):
    pltpu.make_async_copy(ones_v, deg_sh.at[idx_v.at[0]], sem).wait()
  plsc.subcore_barrier()
  pltpu.sync_copy(deg_sh.at[pl.ds(zb, RPT)], stage_v)
  pltpu.sync_copy(stage_v, out_hbm.at[pl.ds(c * NP + zb, RPT)])


_deg_kernel = pl.kernel(
    _deg_body,
    out_type=jax.ShapeDtypeStruct((NC * NP, 16), jnp.float32),
    mesh=_sc_mesh,
    compiler_params=_sc_params,
    scratch_types=[
        pltpu.VMEM((CH, 16), jnp.float32),
        pltpu.VMEM((RPT, 16), jnp.float32),
        pltpu.VMEM((_DCH, CH), jnp.int32),
        pltpu.SemaphoreType.DMA,
        pltpu.VMEM_SHARED((NP, 16), jnp.float32),
    ],
)


_MCH = EPT // CH


_MK = 10
_NBLK = _MCH // _MK


def _msg_body(g_hbm, src2d_hbm, dst2d_hbm, out_hbm, rows_v, sidx_v,
              didx_v, semg, sems, acc_sh, g_sh):
  c = lax.axis_index("c")
  s = lax.axis_index("s")
  zb = s * RPT
  z0 = rows_v.at[0]
  _fill(z0, CH, HALF, 0.0)
  for k in range(RPT // CH):
    pltpu.async_copy(z0, acc_sh.at[pl.ds(zb + k * CH, CH)], sems)
  for k in range(RPT // CH):
    pltpu.make_async_copy(z0, acc_sh.at[pl.ds(zb, CH)], sems).wait()
  pltpu.async_copy(g_hbm.at[pl.ds(c * NP + zb, RPT)],
                   g_sh.at[pl.ds(zb, RPT)], sems)
  pltpu.sync_copy(src2d_hbm.at[pl.ds(s * _MCH, _MCH)], sidx_v)
  pltpu.sync_copy(dst2d_hbm.at[pl.ds(s * _MCH, _MCH)], didx_v)
  pltpu.make_async_copy(g_hbm.at[pl.ds(c * NP + zb, RPT)],
                        g_sh.at[pl.ds(zb, RPT)], sems).wait()
  plsc.subcore_barrier()
  @pl.loop(0, _NBLK)
  def _(i):
    j = i * _MK
    for k in range(_MK):
      @pl.when(i > 0)
      def _():
        pltpu.make_async_copy(rows_v.at[k], acc_sh.at[didx_v.at[i]],
                              sems).wait()
      pltpu.async_copy(g_sh.at[sidx_v.at[j + k]], rows_v.at[k], semg[k])
    for k in range(_MK):
      pltpu.make_async_copy(g_sh.at[sidx_v.at[j + k]], rows_v.at[k],
                            semg[k]).wait()
      pltpu.async_copy(rows_v.at[k], acc_sh.at[didx_v.at[j + k]], sems,
                       add=True)
  for k in range(_MK):
    pltpu.make_async_copy(rows_v.at[k], acc_sh.at[didx_v.at[0]], sems).wait()
  plsc.subcore_barrier()
  pltpu.sync_copy(acc_sh.at[pl.ds(zb, RPT)],
                  out_hbm.at[pl.ds(c * NP + zb, RPT)])


_msg_kernel = pl.kernel(
    _msg_body,
    out_type=jax.ShapeDtypeStruct((NC * NP, HALF), jnp.float32),
    mesh=_sc_mesh,
    compiler_params=_sc_params,
    scratch_types=[
        pltpu.VMEM((_MK, CH, HALF), jnp.float32),
        pltpu.VMEM((_MCH, CH), jnp.int32),
        pltpu.VMEM((_MCH, CH), jnp.int32),
        [pltpu.SemaphoreType.DMA] * _MK,
        pltpu.SemaphoreType.DMA,
        pltpu.VMEM_SHARED((NP, HALF), jnp.float32),
        pltpu.VMEM_SHARED((NP, HALF), jnp.float32),
    ],
)


_TC_R = 1280


def _dis_of(deg_ref):
  deg = deg_ref[0, :, 0:1] + deg_ref[1, :, 0:1] + 1.0
  return lax.rsqrt(deg)


def _pre_body(deg_ref, x_ref, w_ref, g_ref):
  dis = _dis_of(deg_ref)
  h = jnp.dot(x_ref[...], w_ref[...], preferred_element_type=jnp.float32)
  g = h * dis
  g_ref[0] = g[:, :HALF]
  g_ref[1] = g[:, HALF:]


def _mid_body(deg_ref, acc_ref, g_ref, b_ref, w_ref, gout_ref):
  dis = _dis_of(deg_ref)
  srow = acc_ref[...] + g_ref[...]
  s64 = jnp.concatenate([srow[0], srow[1]], axis=1)
  y = s64 * dis + b_ref[...]
  h = jnp.dot(y, w_ref[...], preferred_element_type=jnp.float32)
  g2 = h * dis
  gout_ref[0] = g2[:, :HALF]
  gout_ref[1] = g2[:, HALF:]


def _fin_body(deg_ref, acc_ref, g_ref, b_ref, wout_ref, bout_ref, y_ref, o_ref):
  dis = _dis_of(deg_ref)
  srow = acc_ref[...] + g_ref[...]
  s64 = jnp.concatenate([srow[0], srow[1]], axis=1)
  y = s64 * dis + b_ref[...]
  y_ref[...] = y
  o_ref[...] = jnp.dot(y, wout_ref[...],
                       preferred_element_type=jnp.float32) + bout_ref[...]


_deg_spec = pl.BlockSpec((2, _TC_R, 16), lambda i: (0, i, 0))
_g_spec = pl.BlockSpec((2, _TC_R, HALF), lambda i: (0, i, 0))


_pre_kernel = pl.pallas_call(
    _pre_body,
    grid=(NP // _TC_R,),
    in_specs=[
        _deg_spec,
        pl.BlockSpec((_TC_R, D_IN), lambda i: (i, 0)),
        pl.BlockSpec((D_IN, EMB), lambda i: (0, 0)),
    ],
    out_specs=_g_spec,
    out_shape=jax.ShapeDtypeStruct((2, NP, HALF), jnp.float32),
)

_mid_kernel = pl.pallas_call(
    _mid_body,
    grid=(NP // _TC_R,),
    in_specs=[
        _deg_spec,
        _g_spec,
        _g_spec,
        pl.BlockSpec((1, EMB), lambda i: (0, 0)),
        pl.BlockSpec((EMB, EMB), lambda i: (0, 0)),
    ],
    out_specs=_g_spec,
    out_shape=jax.ShapeDtypeStruct((2, NP, HALF), jnp.float32),
)

_fin_kernel = pl.pallas_call(
    _fin_body,
    grid=(NP // _TC_R,),
    in_specs=[
        _deg_spec,
        _g_spec,
        _g_spec,
        pl.BlockSpec((1, EMB), lambda i: (0, 0)),
        pl.BlockSpec((EMB, 1), lambda i: (0, 0)),
        pl.BlockSpec((1, 1), lambda i: (0, 0)),
    ],
    out_specs=[
        pl.BlockSpec((_TC_R, EMB), lambda i: (i, 0)),
        pl.BlockSpec((_TC_R, 1), lambda i: (i, 0)),
    ],
    out_shape=[
        jax.ShapeDtypeStruct((NP, EMB), jnp.float32),
        jax.ShapeDtypeStruct((NP, 1), jnp.float32),
    ],
)


def kernel(x, edge_index, batch_index, W0, b0, W1, b1, W2, b2, W3, b3,
           Wout, bout):
  del batch_index
  pad = EP - E
  padv = jnp.full((pad,), N, jnp.int32)
  src_p = jnp.concatenate([edge_index[0], padv])
  dst_p = jnp.concatenate([edge_index[1], padv])
  srcl = src_p.reshape(EP // CH, CH)
  dst2 = dst_p.reshape(EP // CH, CH)
  x_p = jnp.pad(x, ((0, NP - N), (0, 0)))

  degtab = _deg_kernel(dst2).reshape(2, NP, 16)
  g = _pre_kernel(degtab, x_p, W0)
  for (b_l, W_next) in ((b0, W1), (b1, W2), (b2, W3)):
    acc = _msg_kernel(g.reshape(NC * NP, HALF), srcl, dst2)
    g = _mid_kernel(degtab, acc.reshape(2, NP, HALF), g,
                    b_l.reshape(1, EMB), W_next)
  acc3 = _msg_kernel(g.reshape(NC * NP, HALF), srcl, dst2)
  y4, out = _fin_kernel(degtab, acc3.reshape(2, NP, HALF), g,
                        b3.reshape(1, EMB), Wout, bout.reshape(1, 1))
  return (out[:N], y4[:N])


# speedup vs baseline: 1.0277x; 1.0277x over previous
import functools

import jax
import jax.numpy as jnp
from jax import lax
from jax.experimental import pallas as pl
from jax.experimental.pallas import tpu as pltpu
from jax.experimental.pallas import tpu_sc as plsc

N = 10000
E = 320000
D_IN = 128
EMB = 64
HALF = EMB // 2

NC, NS = 2, 16
NW = NC * NS
NP = 10240
EP = 327680
CH = 128
RPT = NP // NS
EPT = EP // NS

_sc_mesh = plsc.VectorSubcoreMesh(
    core_axis_name="c", subcore_axis_name="s", num_cores=NC, num_subcores=NS)
_sc_params = pltpu.CompilerParams(use_tc_tiling_on_sc=False)


def _fill(ref, rows, cols, value):
  @pl.loop(0, rows)
  def _(r):
    for k in range(cols // 16):
      ref[r, pl.ds(k * 16, 16)] = jnp.full((16,), value, jnp.float32)


_DCH = EP // NW // CH
_K = 8


def _deg_body(dst2d_hbm, out_hbm, ones_v, stage_v, idx_v, sem, deg_sh):
  c = lax.axis_index("c")
  s = lax.axis_index("s")
  wid = c * NS + s
  zb = s * RPT
  pltpu.async_copy(dst2d_hbm.at[pl.ds(wid * _DCH, _DCH)], idx_v, sem)
  _fill(ones_v, CH, 16, 1.0)
  _fill(stage_v.at[pl.ds(0, CH)], CH, 16, 0.0)
  for k in range(RPT // CH):
    pltpu.async_copy(stage_v.at[pl.ds(0, CH)],
                     deg_sh.at[pl.ds(zb + k * CH, CH)], sem)
  pltpu.make_async_copy(dst2d_hbm.at[pl.ds(wid * _DCH, _DCH)], idx_v,
                        sem).wait()
  for k in range(RPT // CH):
    pltpu.make_async_copy(stage_v.at[pl.ds(0, CH)],
                          deg_sh.at[pl.ds(zb, CH)], sem).wait()
  plsc.subcore_barrier()
  @pl.loop(0, _DCH)
  def _(j):
    @pl.when(j >= _K)
    def _():
      pltpu.make_async_copy(ones_v, deg_sh.at[idx_v.at[j]], sem).wait()
    pltpu.async_copy(ones_v, deg_sh.at[idx_v.at[j]], sem, add=True)
  for _ in range(_K):
    pltpu.make_async_copy(ones_v, deg_sh.at[idx_v.at[0]], sem).wait()
  plsc.subcore_barrier()
  pltpu.sync_copy(deg_sh.at[pl.ds(zb, RPT)], stage_v)
  pltpu.sync_copy(stage_v, out_hbm.at[pl.ds(c * NP + zb, RPT)])


_deg_kernel = pl.kernel(
    _deg_body,
    out_type=jax.ShapeDtypeStruct((NC * NP, 16), jnp.float32),
    mesh=_sc_mesh,
    compiler_params=_sc_params,
    scratch_types=[
        pltpu.VMEM((CH, 16), jnp.float32),
        pltpu.VMEM((RPT, 16), jnp.float32),
        pltpu.VMEM((_DCH, CH), jnp.int32),
        pltpu.SemaphoreType.DMA,
        pltpu.VMEM_SHARED((NP, 16), jnp.float32),
    ],
)


_MCH = EPT // CH


_MK = 10
_NBLK = _MCH // _MK


def _msg_body(g_hbm, src2d_hbm, dst2d_hbm, out_hbm, rows_v, sidx_v,
              didx_v, semg, sems, acc_sh, g_sh):
  c = lax.axis_index("c")
  s = lax.axis_index("s")
  zb = s * RPT
  pltpu.async_copy(g_hbm.at[pl.ds(c * NP + zb, RPT)],
                   g_sh.at[pl.ds(zb, RPT)], semg[1])
  pltpu.async_copy(src2d_hbm.at[pl.ds(s * _MCH, _MCH)], sidx_v, semg[2])
  pltpu.async_copy(dst2d_hbm.at[pl.ds(s * _MCH, _MCH)], didx_v, semg[3])
  z0 = rows_v.at[0]
  _fill(z0, CH, HALF, 0.0)
  for k in range(RPT // CH):
    pltpu.async_copy(z0, acc_sh.at[pl.ds(zb + k * CH, CH)], sems)
  for k in range(RPT // CH):
    pltpu.make_async_copy(z0, acc_sh.at[pl.ds(zb, CH)], sems).wait()
  pltpu.make_async_copy(g_hbm.at[pl.ds(c * NP + zb, RPT)],
                        g_sh.at[pl.ds(zb, RPT)], semg[1]).wait()
  pltpu.make_async_copy(src2d_hbm.at[pl.ds(s * _MCH, _MCH)], sidx_v,
                        semg[2]).wait()
  pltpu.make_async_copy(dst2d_hbm.at[pl.ds(s * _MCH, _MCH)], didx_v,
                        semg[3]).wait()
  plsc.subcore_barrier()
  @pl.loop(0, _NBLK)
  def _(i):
    j = i * _MK
    for k in range(_MK):
      @pl.when(i > 0)
      def _():
        pltpu.make_async_copy(rows_v.at[k], acc_sh.at[didx_v.at[i]],
                              sems).wait()
      pltpu.async_copy(g_sh.at[sidx_v.at[j + k]], rows_v.at[k], semg[k])
    for k in range(_MK):
      pltpu.make_async_copy(g_sh.at[sidx_v.at[j + k]], rows_v.at[k],
                            semg[k]).wait()
      pltpu.async_copy(rows_v.at[k], acc_sh.at[didx_v.at[j + k]], sems,
                       add=True)
  for k in range(_MK):
    pltpu.make_async_copy(rows_v.at[k], acc_sh.at[didx_v.at[0]], sems).wait()
  plsc.subcore_barrier()
  pltpu.sync_copy(acc_sh.at[pl.ds(zb, RPT)],
                  out_hbm.at[pl.ds(c * NP + zb, RPT)])


_msg_kernel = pl.kernel(
    _msg_body,
    out_type=jax.ShapeDtypeStruct((NC * NP, HALF), jnp.float32),
    mesh=_sc_mesh,
    compiler_params=_sc_params,
    scratch_types=[
        pltpu.VMEM((_MK, CH, HALF), jnp.float32),
        pltpu.VMEM((_MCH, CH), jnp.int32),
        pltpu.VMEM((_MCH, CH), jnp.int32),
        [pltpu.SemaphoreType.DMA] * _MK,
        pltpu.SemaphoreType.DMA,
        pltpu.VMEM_SHARED((NP, HALF), jnp.float32),
        pltpu.VMEM_SHARED((NP, HALF), jnp.float32),
    ],
)


_TC_R = 1280


def _dis_of(deg_ref):
  deg = deg_ref[0, :, 0:1] + deg_ref[1, :, 0:1] + 1.0
  return lax.rsqrt(deg)


def _pre_body(deg_ref, x_ref, w_ref, g_ref):
  dis = _dis_of(deg_ref)
  h = jnp.dot(x_ref[...], w_ref[...], preferred_element_type=jnp.float32)
  g = h * dis
  g_ref[0] = g[:, :HALF]
  g_ref[1] = g[:, HALF:]


def _mid_body(deg_ref, acc_ref, g_ref, b_ref, w_ref, gout_ref):
  dis = _dis_of(deg_ref)
  srow = acc_ref[...] + g_ref[...]
  s64 = jnp.concatenate([srow[0], srow[1]], axis=1)
  y = s64 * dis + b_ref[...]
  h = jnp.dot(y, w_ref[...], preferred_element_type=jnp.float32)
  g2 = h * dis
  gout_ref[0] = g2[:, :HALF]
  gout_ref[1] = g2[:, HALF:]


def _fin_body(deg_ref, acc_ref, g_ref, b_ref, wout_ref, bout_ref, y_ref, o_ref):
  dis = _dis_of(deg_ref)
  srow = acc_ref[...] + g_ref[...]
  s64 = jnp.concatenate([srow[0], srow[1]], axis=1)
  y = s64 * dis + b_ref[...]
  y_ref[...] = y
  o_ref[...] = jnp.dot(y, wout_ref[...],
                       preferred_element_type=jnp.float32) + bout_ref[...]


_deg_spec = pl.BlockSpec((2, _TC_R, 16), lambda i: (0, i, 0))
_g_spec = pl.BlockSpec((2, _TC_R, HALF), lambda i: (0, i, 0))


_pre_kernel = pl.pallas_call(
    _pre_body,
    grid=(NP // _TC_R,),
    in_specs=[
        _deg_spec,
        pl.BlockSpec((_TC_R, D_IN), lambda i: (i, 0)),
        pl.BlockSpec((D_IN, EMB), lambda i: (0, 0)),
    ],
    out_specs=_g_spec,
    out_shape=jax.ShapeDtypeStruct((2, NP, HALF), jnp.float32),
)

_mid_kernel = pl.pallas_call(
    _mid_body,
    grid=(NP // _TC_R,),
    in_specs=[
        _deg_spec,
        _g_spec,
        _g_spec,
        pl.BlockSpec((1, EMB), lambda i: (0, 0)),
        pl.BlockSpec((EMB, EMB), lambda i: (0, 0)),
    ],
    out_specs=_g_spec,
    out_shape=jax.ShapeDtypeStruct((2, NP, HALF), jnp.float32),
)

_fin_kernel = pl.pallas_call(
    _fin_body,
    grid=(NP // _TC_R,),
    in_specs=[
        _deg_spec,
        _g_spec,
        _g_spec,
        pl.BlockSpec((1, EMB), lambda i: (0, 0)),
        pl.BlockSpec((EMB, 1), lambda i: (0, 0)),
        pl.BlockSpec((1, 1), lambda i: (0, 0)),
    ],
    out_specs=[
        pl.BlockSpec((_TC_R, EMB), lambda i: (i, 0)),
        pl.BlockSpec((_TC_R, 1), lambda i: (i, 0)),
    ],
    out_shape=[
        jax.ShapeDtypeStruct((NP, EMB), jnp.float32),
        jax.ShapeDtypeStruct((NP, 1), jnp.float32),
    ],
)


def kernel(x, edge_index, batch_index, W0, b0, W1, b1, W2, b2, W3, b3,
           Wout, bout):
  del batch_index
  pad = EP - E
  padv = jnp.full((pad,), N, jnp.int32)
  src_p = jnp.concatenate([edge_index[0], padv])
  dst_p = jnp.concatenate([edge_index[1], padv])
  srcl = src_p.reshape(EP // CH, CH)
  dst2 = dst_p.reshape(EP // CH, CH)
  x_p = jnp.pad(x, ((0, NP - N), (0, 0)))

  degtab = _deg_kernel(dst2).reshape(2, NP, 16)
  g = _pre_kernel(degtab, x_p, W0)
  for (b_l, W_next) in ((b0, W1), (b1, W2), (b2, W3)):
    acc = _msg_kernel(g.reshape(NC * NP, HALF), srcl, dst2)
    g = _mid_kernel(degtab, acc.reshape(2, NP, HALF), g,
                    b_l.reshape(1, EMB), W_next)
  acc3 = _msg_kernel(g.reshape(NC * NP, HALF), srcl, dst2)
  y4, out = _fin_kernel(degtab, acc3.reshape(2, NP, HALF), g,
                        b3.reshape(1, EMB), Wout, bout.reshape(1, 1))
  return (out[:N], y4[:N])


# speedup vs baseline: 1.0287x; 1.0010x over previous
import jax
import jax.numpy as jnp
from jax import lax
from jax.experimental import pallas as pl
from jax.experimental.pallas import tpu as pltpu
from jax.experimental.pallas import tpu_sc as plsc

N = 10000
E = 320000
D_IN = 128
EMB = 64
HALF = EMB // 2

NC, NS = 2, 16
NW = NC * NS
NP = 10240
EP = 327680
CH = 128
RPT = NP // NS
EPT = EP // NS

_sc_mesh = plsc.VectorSubcoreMesh(
    core_axis_name="c", subcore_axis_name="s", num_cores=NC, num_subcores=NS)
_sc_params = pltpu.CompilerParams(use_tc_tiling_on_sc=False)


def _fill(ref, rows, cols, value):
  @pl.loop(0, rows)
  def _(r):
    for k in range(cols // 16):
      ref[r, pl.ds(k * 16, 16)] = jnp.full((16,), value, jnp.float32)


_DCH = EP // NW // CH
_K = 8


def _deg_body(dst2d_hbm, out_hbm, ones_v, stage_v, idx_v, sem, deg_sh):
  c = lax.axis_index("c")
  s = lax.axis_index("s")
  wid = c * NS + s
  zb = s * RPT
  pltpu.async_copy(dst2d_hbm.at[pl.ds(wid * _DCH, _DCH)], idx_v, sem)
  _fill(ones_v, CH, 16, 1.0)
  _fill(stage_v.at[pl.ds(0, CH)], CH, 16, 0.0)
  for k in range(RPT // CH):
    pltpu.async_copy(stage_v.at[pl.ds(0, CH)],
                     deg_sh.at[pl.ds(zb + k * CH, CH)], sem)
  pltpu.make_async_copy(dst2d_hbm.at[pl.ds(wid * _DCH, _DCH)], idx_v,
                        sem).wait()
  for k in range(RPT // CH):
    pltpu.make_async_copy(stage_v.at[pl.ds(0, CH)],
                          deg_sh.at[pl.ds(zb, CH)], sem).wait()
  plsc.subcore_barrier()
  @pl.loop(0, _DCH)
  def _(j):
    @pl.when(j >= _K)
    def _():
      pltpu.make_async_copy(ones_v, deg_sh.at[idx_v.at[j]], sem).wait()
    pltpu.async_copy(ones_v, deg_sh.at[idx_v.at[j]], sem, add=True)
  for _ in range(_K):
    pltpu.make_async_copy(ones_v, deg_sh.at[idx_v.at[0]], sem).wait()
  plsc.subcore_barrier()
  pltpu.sync_copy(deg_sh.at[pl.ds(zb, RPT)], stage_v)
  pltpu.sync_copy(stage_v, out_hbm.at[pl.ds(c * NP + zb, RPT)])


_deg_kernel = pl.kernel(
    _deg_body,
    out_type=jax.ShapeDtypeStruct((NC * NP, 16), jnp.float32),
    mesh=_sc_mesh,
    compiler_params=_sc_params,
    scratch_types=[
        pltpu.VMEM((CH, 16), jnp.float32),
        pltpu.VMEM((RPT, 16), jnp.float32),
        pltpu.VMEM((_DCH, CH), jnp.int32),
        pltpu.SemaphoreType.DMA,
        pltpu.VMEM_SHARED((NP, 16), jnp.float32),
    ],
)


_MCH = EPT // CH


_MK = 10
_NBLK = _MCH // _MK


def _msg_body(g_hbm, src2d_hbm, dst2d_hbm, out_hbm, rows_v, sidx_v,
              didx_v, semg, sems, acc_sh, g_sh):
  c = lax.axis_index("c")
  s = lax.axis_index("s")
  zb = s * RPT
  pltpu.async_copy(g_hbm.at[pl.ds(c * NP + zb, RPT)],
                   g_sh.at[pl.ds(zb, RPT)], semg[1])
  pltpu.async_copy(src2d_hbm.at[pl.ds(s * _MCH, _MCH)], sidx_v, semg[2])
  pltpu.async_copy(dst2d_hbm.at[pl.ds(s * _MCH, _MCH)], didx_v, semg[3])
  z0 = rows_v.at[0]
  _fill(z0, CH, HALF, 0.0)
  for k in range(RPT // CH):
    pltpu.async_copy(z0, acc_sh.at[pl.ds(zb + k * CH, CH)], sems)
  for k in range(RPT // CH):
    pltpu.make_async_copy(z0, acc_sh.at[pl.ds(zb, CH)], sems).wait()
  pltpu.make_async_copy(g_hbm.at[pl.ds(c * NP + zb, RPT)],
                        g_sh.at[pl.ds(zb, RPT)], semg[1]).wait()
  pltpu.make_async_copy(src2d_hbm.at[pl.ds(s * _MCH, _MCH)], sidx_v,
                        semg[2]).wait()
  pltpu.make_async_copy(dst2d_hbm.at[pl.ds(s * _MCH, _MCH)], didx_v,
                        semg[3]).wait()
  plsc.subcore_barrier()
  @pl.loop(0, _NBLK)
  def _(i):
    j = i * _MK
    for k in range(_MK):
      @pl.when(i > 0)
      def _():
        pltpu.make_async_copy(rows_v.at[k], acc_sh.at[didx_v.at[i]],
                              sems).wait()
      pltpu.async_copy(g_sh.at[sidx_v.at[j + k]], rows_v.at[k], semg[k])
    for k in range(_MK):
      pltpu.make_async_copy(g_sh.at[sidx_v.at[j + k]], rows_v.at[k],
                            semg[k]).wait()
      pltpu.async_copy(rows_v.at[k], acc_sh.at[didx_v.at[j + k]], sems,
                       add=True)
  for k in range(_MK):
    pltpu.make_async_copy(rows_v.at[k], acc_sh.at[didx_v.at[0]], sems).wait()
  plsc.subcore_barrier()
  pltpu.sync_copy(acc_sh.at[pl.ds(zb, RPT)],
                  out_hbm.at[pl.ds(c * NP + zb, RPT)])


_msg_kernel = pl.kernel(
    _msg_body,
    out_type=jax.ShapeDtypeStruct((NC * NP, HALF), jnp.float32),
    mesh=_sc_mesh,
    compiler_params=_sc_params,
    scratch_types=[
        pltpu.VMEM((_MK, CH, HALF), jnp.float32),
        pltpu.VMEM((_MCH, CH), jnp.int32),
        pltpu.VMEM((_MCH, CH), jnp.int32),
        [pltpu.SemaphoreType.DMA] * _MK,
        pltpu.SemaphoreType.DMA,
        pltpu.VMEM_SHARED((NP, HALF), jnp.float32),
        pltpu.VMEM_SHARED((NP, HALF), jnp.float32),
    ],
)


_TC_R = 1280


def _dis_of(deg_ref):
  deg = deg_ref[0, :, 0:1] + deg_ref[1, :, 0:1] + 1.0
  return lax.rsqrt(deg)


def _pre_body(deg_ref, x_ref, w_ref, g_ref):
  dis = _dis_of(deg_ref)
  h = jnp.dot(x_ref[...], w_ref[...], preferred_element_type=jnp.float32)
  g = h * dis
  g_ref[0] = g[:, :HALF]
  g_ref[1] = g[:, HALF:]


def _mid_body(deg_ref, acc_ref, g_ref, b_ref, w_ref, gout_ref):
  dis = _dis_of(deg_ref)
  srow = acc_ref[...] + g_ref[...]
  s64 = jnp.concatenate([srow[0], srow[1]], axis=1)
  y = s64 * dis + b_ref[...]
  h = jnp.dot(y, w_ref[...], preferred_element_type=jnp.float32)
  g2 = h * dis
  gout_ref[0] = g2[:, :HALF]
  gout_ref[1] = g2[:, HALF:]


def _fin_body(deg_ref, acc_ref, g_ref, b_ref, wout_ref, bout_ref, y_ref, o_ref):
  dis = _dis_of(deg_ref)
  srow = acc_ref[...] + g_ref[...]
  s64 = jnp.concatenate([srow[0], srow[1]], axis=1)
  y = s64 * dis + b_ref[...]
  y_ref[...] = y
  o_ref[...] = jnp.dot(y, wout_ref[...],
                       preferred_element_type=jnp.float32) + bout_ref[...]


_deg_spec = pl.BlockSpec((2, _TC_R, 16), lambda i: (0, i, 0))
_g_spec = pl.BlockSpec((2, _TC_R, HALF), lambda i: (0, i, 0))


_pre_kernel = pl.pallas_call(
    _pre_body,
    grid=(NP // _TC_R,),
    in_specs=[
        _deg_spec,
        pl.BlockSpec((_TC_R, D_IN), lambda i: (i, 0)),
        pl.BlockSpec((D_IN, EMB), lambda i: (0, 0)),
    ],
    out_specs=_g_spec,
    out_shape=jax.ShapeDtypeStruct((2, NP, HALF), jnp.float32),
)

_mid_kernel = pl.pallas_call(
    _mid_body,
    grid=(NP // _TC_R,),
    in_specs=[
        _deg_spec,
        _g_spec,
        _g_spec,
        pl.BlockSpec((1, EMB), lambda i: (0, 0)),
        pl.BlockSpec((EMB, EMB), lambda i: (0, 0)),
    ],
    out_specs=_g_spec,
    out_shape=jax.ShapeDtypeStruct((2, NP, HALF), jnp.float32),
)

_fin_kernel = pl.pallas_call(
    _fin_body,
    grid=(NP // _TC_R,),
    in_specs=[
        _deg_spec,
        _g_spec,
        _g_spec,
        pl.BlockSpec((1, EMB), lambda i: (0, 0)),
        pl.BlockSpec((EMB, 1), lambda i: (0, 0)),
        pl.BlockSpec((1, 1), lambda i: (0, 0)),
    ],
    out_specs=[
        pl.BlockSpec((_TC_R, EMB), lambda i: (i, 0)),
        pl.BlockSpec((_TC_R, 1), lambda i: (i, 0)),
    ],
    out_shape=[
        jax.ShapeDtypeStruct((NP, EMB), jnp.float32),
        jax.ShapeDtypeStruct((NP, 1), jnp.float32),
    ],
)


def kernel(x, edge_index, batch_index, W0, b0, W1, b1, W2, b2, W3, b3,
           Wout, bout):
  del batch_index
  pad = EP - E
  padv = jnp.full((pad,), N, jnp.int32)
  src_p = jnp.concatenate([edge_index[0], padv])
  dst_p = jnp.concatenate([edge_index[1], padv])
  srcl = src_p.reshape(EP // CH, CH)
  dst2 = dst_p.reshape(EP // CH, CH)
  x_p = jnp.pad(x, ((0, NP - N), (0, 0)))

  degtab = _deg_kernel(dst2).reshape(2, NP, 16)
  g = _pre_kernel(degtab, x_p, W0)
  for (b_l, W_next) in ((b0, W1), (b1, W2), (b2, W3)):
    acc = _msg_kernel(g.reshape(NC * NP, HALF), srcl, dst2)
    g = _mid_kernel(degtab, acc.reshape(2, NP, HALF), g,
                    b_l.reshape(1, EMB), W_next)
  acc3 = _msg_kernel(g.reshape(NC * NP, HALF), srcl, dst2)
  y4, out = _fin_kernel(degtab, acc3.reshape(2, NP, HALF), g,
                        b3.reshape(1, EMB), Wout, bout.reshape(1, 1))
  return (out[:N], y4[:N])
